# Initial kernel scaffold; baseline (speedup 1.0000x reference)
import jax
import jax.numpy as jnp
from jax.experimental import pallas as pl


def kernel(node, edge, edge_index, Wn, bn, We, be, Wh, bh, Wa, ba, Wm, bm, Wp1, bp1, Wp2, bp2):
    raise NotImplementedError("write your pallas kernel here")


# speedup vs baseline: 2.0863x; 2.0863x over previous
import functools

import jax
import jax.numpy as jnp
from jax import lax
from jax.experimental import pallas as pl
from jax.experimental.pallas import tpu as pltpu
from jax.experimental.pallas import tpu_sc as plsc

E = 320000
N = 10000
H = 128
CH = 128
NCHUNK = E // CH
NC, NS = 2, 16
NW = NC * NS
TRIPS = -(-NCHUNK // NW)
NP = 10240
ROWS_PER_TILE = NP // NS

_f32 = jnp.float32


@functools.partial(
    pl.kernel,
    mesh=plsc.VectorSubcoreMesh(core_axis_name="c", subcore_axis_name="s"),
    out_type=[
        jax.ShapeDtypeStruct((E, H), _f32),
        jax.ShapeDtypeStruct((NC * NP, H), _f32),
    ],
    scratch_types=[
        pltpu.VMEM((CH, H), _f32),
        pltpu.VMEM((CH, H), _f32),
        pltpu.VMEM((CH,), jnp.int32),
        pltpu.VMEM((CH,), jnp.int32),
        pltpu.VMEM_SHARED((NP, H), _f32),
        pltpu.SemaphoreType.DMA,
    ],
)
def _sc_pass(a_hbm, gidx_hbm, sidx_hbm, table_hbm, zeros_hbm,
             h_out, part_out, a_v, g_v, gi_v, si_v, acc_sh, sem):
    c = lax.axis_index("c")
    s = lax.axis_index("s")
    w = s * NC + c

    pltpu.sync_copy(zeros_hbm.at[pl.ds(s * ROWS_PER_TILE, ROWS_PER_TILE)],
                    acc_sh.at[pl.ds(s * ROWS_PER_TILE, ROWS_PER_TILE)])
    plsc.subcore_barrier()

    def chunk_body(j, carry):
        cid = w + NW * j

        @pl.when(cid < NCHUNK)
        def _():
            base = cid * CH
            pltpu.sync_copy(a_hbm.at[pl.ds(base, CH)], a_v)
            pltpu.sync_copy(gidx_hbm.at[pl.ds(base, CH)], gi_v)
            pltpu.sync_copy(sidx_hbm.at[pl.ds(base, CH)], si_v)
            pltpu.async_copy(table_hbm.at[gi_v], g_v, sem).wait()

            def row_body(r, rc):
                for q in range(H // 16):
                    sl = pl.ds(q * 16, 16)
                    a_v[r, sl] = jnp.maximum(a_v[r, sl] + g_v[r, sl], 0.0)
                return rc

            lax.fori_loop(0, CH, row_body, 0)
            pltpu.sync_copy(a_v, h_out.at[pl.ds(base, CH)])
            pltpu.sync_copy(a_v, acc_sh.at[si_v], add=True)

        return carry

    lax.fori_loop(0, TRIPS, chunk_body, 0)
    plsc.subcore_barrier()
    pltpu.sync_copy(acc_sh.at[pl.ds(s * ROWS_PER_TILE, ROWS_PER_TILE)],
                    part_out.at[pl.ds(c * NP + s * ROWS_PER_TILE, ROWS_PER_TILE)])


def _dot(a, b):
    return jnp.dot(a, b, preferred_element_type=_f32)


def _leaky(x):
    return jnp.where(x > 0, x, 0.01 * x)


def _prep_nodes_body(node_ref, Wn_ref, bn_ref, Wh1_ref, n_ref, nWh_ref):
    n = _leaky(_dot(node_ref[...], Wn_ref[...]) + bn_ref[...])
    n_ref[...] = n
    nWh_ref[...] = _dot(n, Wh1_ref[...])


def _prep_edges_body(edge_ref, We_ref, be_ref, Wh2_ref, bh_ref, eW_ref):
    e = _leaky(_dot(edge_ref[...], We_ref[...]) + be_ref[...])
    eW_ref[...] = _dot(e, Wh2_ref[...]) + bh_ref[...]


def _layer_body(inp_ref, h_ref, Wm_ref, d_ref):
    d_ref[...] = inp_ref[...] - _dot(h_ref[...], Wm_ref[...])


def _fullw_body(p0_ref, p1_ref, Wm_ref, bm_ref, fw_ref):
    fw_ref[...] = _dot(p0_ref[...] + p1_ref[...], Wm_ref[...]) + bm_ref[...]


def _readout_body(n_ref, p0_ref, p1_ref, Wa1_ref, Wa2_ref, ba_ref,
                  Wp1_ref, bp1_ref, Wp2_ref, bp2_ref, out_ref, acc_ref):
    i = pl.program_id(0)

    @pl.when(i == 0)
    def _():
        acc_ref[...] = jnp.zeros_like(acc_ref)

    agg = p0_ref[...] + p1_ref[...]
    hid = jnp.maximum(_dot(n_ref[...], Wa1_ref[...]) + _dot(agg, Wa2_ref[...])
                      + ba_ref[...], 0.0)
    acc_ref[0:1, :] += jnp.sum(hid, axis=0, keepdims=True)

    @pl.when(i == pl.num_programs(0) - 1)
    def _():
        g = acc_ref[0:1, :]
        z = jnp.maximum(_dot(g, Wp1_ref[...]) + bp1_ref[...], 0.0)
        out_ref[...] = _dot(z, Wp2_ref[...]) + bp2_ref[...]


def _full_spec(shape):
    return pl.BlockSpec(shape, lambda i: (0, 0))


def _prep_nodes(node, Wn, bn, Wh1):
    blk = 1000
    return pl.pallas_call(
        _prep_nodes_body,
        grid=(N // blk,),
        in_specs=[
            pl.BlockSpec((blk, H), lambda i: (i, 0)),
            _full_spec((H, H)), _full_spec((1, H)), _full_spec((H, H)),
        ],
        out_specs=[pl.BlockSpec((blk, H), lambda i: (i, 0))] * 2,
        out_shape=[jax.ShapeDtypeStruct((N, H), _f32)] * 2,
    )(node, Wn, bn, Wh1)


def _prep_edges(edge, We, be, Wh2, bh):
    blk = 2000
    nb = edge.shape[1]
    return pl.pallas_call(
        _prep_edges_body,
        grid=(E // blk,),
        in_specs=[
            pl.BlockSpec((blk, nb), lambda i: (i, 0)),
            _full_spec((nb, H)), _full_spec((1, H)),
            _full_spec((H, H)), _full_spec((1, H)),
        ],
        out_specs=pl.BlockSpec((blk, H), lambda i: (i, 0)),
        out_shape=jax.ShapeDtypeStruct((E, H), _f32),
    )(edge, We, be, Wh2, bh)


def _layer_tc(inputs, h, Wmi):
    blk = 2000
    return pl.pallas_call(
        _layer_body,
        grid=(E // blk,),
        in_specs=[
            pl.BlockSpec((blk, H), lambda i: (i, 0)),
            pl.BlockSpec((blk, H), lambda i: (i, 0)),
            _full_spec((H, H)),
        ],
        out_specs=pl.BlockSpec((blk, H), lambda i: (i, 0)),
        out_shape=jax.ShapeDtypeStruct((E, H), _f32),
    )(inputs, h, Wmi)


def _fullw_tc(parts, Wmi, bmi):
    blk = 80
    return pl.pallas_call(
        _fullw_body,
        grid=(N // blk,),
        in_specs=[
            pl.BlockSpec((blk, H), lambda i: (i, 0)),
            pl.BlockSpec((blk, H), lambda i: (i + NP // blk, 0)),
            _full_spec((H, H)), _full_spec((1, H)),
        ],
        out_specs=pl.BlockSpec((blk, H), lambda i: (i, 0)),
        out_shape=jax.ShapeDtypeStruct((N, H), _f32),
    )(parts, parts, Wmi, bmi)


def _readout(n, parts, Wa1, Wa2, ba, Wp1, bp1, Wp2, bp2):
    blk = 80
    h2 = H // 2
    return pl.pallas_call(
        _readout_body,
        grid=(N // blk,),
        in_specs=[
            pl.BlockSpec((blk, H), lambda i: (i, 0)),
            pl.BlockSpec((blk, H), lambda i: (i, 0)),
            pl.BlockSpec((blk, H), lambda i: (i + NP // blk, 0)),
            _full_spec((H, H)), _full_spec((H, H)), _full_spec((1, H)),
            _full_spec((H, h2)), _full_spec((1, h2)),
            _full_spec((h2, 1)), _full_spec((1, 1)),
        ],
        out_specs=pl.BlockSpec((1, 1), lambda i: (0, 0)),
        out_shape=jax.ShapeDtypeStruct((1, 1), _f32),
        scratch_shapes=[pltpu.VMEM((8, H), _f32)],
    )(n, parts, parts, Wa1, Wa2, ba, Wp1, bp1, Wp2, bp2)


def kernel(node, edge, edge_index, Wn, bn, We, be, Wh, bh, Wa, ba,
           Wm, bm, Wp1, bp1, Wp2, bp2):
    src = edge_index[0]
    dst = edge_index[1]
    Wh1, Wh2 = Wh[:H], Wh[H:]
    Wa1, Wa2 = Wa[:H], Wa[H:]
    bn_, be_, bh_, ba_, bp1_, bp2_ = (
        x.reshape(1, -1) for x in (bn, be, bh, ba, bp1, bp2))
    zeros = jnp.zeros((NP, H), _f32)

    n, nWh = _prep_nodes(node, Wn, bn_, Wh1)
    eW = _prep_edges(edge, We, be_, Wh2, bh_)
    h, parts = _sc_pass(eW, src, dst, nWh, zeros)
    inputs = h
    for i in range(3):
        fullW = _fullw_tc(parts, Wm[i], bm[i].reshape(1, -1))
        d = _layer_tc(inputs, h, Wm[i])
        h, parts = _sc_pass(d, dst, dst, fullW, zeros)
    return _readout(n, parts, Wa1, Wa2, ba_, Wp1, bp1_, Wp2, bp2_)


# speedup vs baseline: 2.9378x; 1.4082x over previous
import functools

import jax
import jax.numpy as jnp
from jax import lax
from jax.experimental import pallas as pl
from jax.experimental.pallas import tpu as pltpu
from jax.experimental.pallas import tpu_sc as plsc

E = 320000
N = 10000
H = 128
CH = 64
NCHUNK = E // CH
NC, NS = 2, 16
NW = NC * NS
CPT = 157
NP = 10240
ROWS_PER_TILE = NP // NS

_f32 = jnp.float32


@functools.partial(
    pl.kernel,
    mesh=plsc.VectorSubcoreMesh(core_axis_name="c", subcore_axis_name="s"),
    out_type=[
        jax.ShapeDtypeStruct((E, H), _f32),
        jax.ShapeDtypeStruct((NC * NP, H), _f32),
    ],
    scratch_types=[
        pltpu.VMEM((CH, H), _f32), pltpu.VMEM((CH, H), _f32),
        pltpu.VMEM((CH, H), _f32), pltpu.VMEM((CH, H), _f32),
        pltpu.VMEM((CH,), jnp.int32), pltpu.VMEM((CH,), jnp.int32),
        pltpu.VMEM((CH,), jnp.int32), pltpu.VMEM((CH,), jnp.int32),
        pltpu.VMEM_SHARED((NP, H), _f32),
        pltpu.SemaphoreType.DMA, pltpu.SemaphoreType.DMA,
        pltpu.SemaphoreType.DMA, pltpu.SemaphoreType.DMA,
        pltpu.SemaphoreType.DMA, pltpu.SemaphoreType.DMA,
        pltpu.SemaphoreType.DMA, pltpu.SemaphoreType.DMA,
        pltpu.SemaphoreType.DMA, pltpu.SemaphoreType.DMA,
        pltpu.SemaphoreType.DMA, pltpu.SemaphoreType.DMA,
    ],
)
def _sc_pass(a_hbm, gidx_hbm, sidx_hbm, table_hbm, zeros_hbm,
             h_out, part_out,
             a0, a1, g0, g1, gi0, gi1, si0, si1, acc_sh,
             sa0, sa1, sg0, sg1, sgi0, sgi1, ssi0, ssi1,
             sh0, sh1, ss0, ss1):
    c = lax.axis_index("c")
    s = lax.axis_index("s")
    w = s * NC + c
    a_v, g_v = (a0, a1), (g0, g1)
    gi_v, si_v = (gi0, gi1), (si0, si1)
    sa, sg = (sa0, sa1), (sg0, sg1)
    sgi, ssi = (sgi0, sgi1), (ssi0, ssi1)
    sh, ss = (sh0, sh1), (ss0, ss1)

    def cid_of(j):
        return w + NW * j

    pltpu.sync_copy(zeros_hbm.at[pl.ds(s * ROWS_PER_TILE, ROWS_PER_TILE)],
                    acc_sh.at[pl.ds(s * ROWS_PER_TILE, ROWS_PER_TILE)])
    plsc.subcore_barrier()

    pltpu.async_copy(gidx_hbm.at[cid_of(0)], gi_v[0], sgi[0])

    @pl.when(cid_of(1) < NCHUNK)
    def _():
        pltpu.async_copy(gidx_hbm.at[cid_of(1)], gi_v[1], sgi[1])

    pltpu.async_copy(sidx_hbm.at[cid_of(0)], si_v[0], ssi[0])
    pltpu.make_async_copy(gidx_hbm.at[cid_of(0)], gi_v[0], sgi[0]).wait()
    pltpu.async_copy(a_hbm.at[pl.ds(cid_of(0) * CH, CH)], a_v[0], sa[0])
    pltpu.async_copy(table_hbm.at[gi_v[0]], g_v[0], sg[0])

    def pair_body(jj, carry):
        for b in range(2):
            j = 2 * jj + b
            o = 1 - b
            cid = cid_of(j)
            cprev = cid_of(j - 1)
            cnext = cid_of(j + 1)

            @pl.when(cid < NCHUNK)
            def _():
                pltpu.make_async_copy(
                    a_hbm.at[pl.ds(cid * CH, CH)], a_v[b], sa[b]).wait()
                pltpu.make_async_copy(
                    table_hbm.at[gi_v[b]], g_v[b], sg[b]).wait()

                def row_body(r, rc):
                    for q in range(H // 16):
                        sl = pl.ds(q * 16, 16)
                        a_v[b][r, sl] = jnp.maximum(
                            a_v[b][r, sl] + g_v[b][r, sl], 0.0)
                    return rc

                lax.fori_loop(0, CH, row_body, 0)
                pltpu.make_async_copy(
                    sidx_hbm.at[cid], si_v[b], ssi[b]).wait()
                pltpu.async_copy(a_v[b], h_out.at[pl.ds(cid * CH, CH)], sh[b])
                pltpu.async_copy(a_v[b], acc_sh.at[si_v[b]], ss[b], add=True)

            @pl.when(jnp.logical_and(j >= 1, cprev < NCHUNK))
            def _():
                pltpu.make_async_copy(
                    a_v[o], h_out.at[pl.ds(cprev * CH, CH)], sh[o]).wait()
                pltpu.make_async_copy(
                    a_v[o], acc_sh.at[si_v[o]], ss[o]).wait()

            @pl.when(cnext < NCHUNK)
            def _():
                pltpu.make_async_copy(
                    gidx_hbm.at[cnext], gi_v[o], sgi[o]).wait()
                pltpu.async_copy(
                    a_hbm.at[pl.ds(cnext * CH, CH)], a_v[o], sa[o])
                pltpu.async_copy(table_hbm.at[gi_v[o]], g_v[o], sg[o])
                pltpu.async_copy(sidx_hbm.at[cnext], si_v[o], ssi[o])

            @pl.when(cid_of(j + 2) < NCHUNK)
            def _():
                pltpu.async_copy(gidx_hbm.at[cid_of(j + 2)], gi_v[b], sgi[b])

        return carry

    lax.fori_loop(0, CPT // 2 + 1, pair_body, 0)
    plsc.subcore_barrier()
    pltpu.sync_copy(acc_sh.at[pl.ds(s * ROWS_PER_TILE, ROWS_PER_TILE)],
                    part_out.at[pl.ds(c * NP + s * ROWS_PER_TILE, ROWS_PER_TILE)])


def _dot(a, b):
    return jnp.dot(a, b, preferred_element_type=_f32)


def _leaky(x):
    return jnp.where(x > 0, x, 0.01 * x)


def _prep_nodes_body(node_ref, Wn_ref, bn_ref, Wh1_ref, n_ref, nWh_ref):
    n = _leaky(_dot(node_ref[...], Wn_ref[...]) + bn_ref[...])
    n_ref[...] = n
    nWh_ref[...] = _dot(n, Wh1_ref[...])


def _prep_edges_body(edge_ref, We_ref, be_ref, Wh2_ref, bh_ref, eW_ref):
    e = _leaky(_dot(edge_ref[...], We_ref[...]) + be_ref[...])
    eW_ref[...] = _dot(e, Wh2_ref[...]) + bh_ref[...]


def _layer_body(inp_ref, h_ref, p0_ref, p1_ref, Wm_ref, bm_ref, d_ref, fw_ref):
    d_ref[...] = inp_ref[...] - _dot(h_ref[...], Wm_ref[...])
    fw_ref[...] = _dot(p0_ref[...] + p1_ref[...], Wm_ref[...]) + bm_ref[...]


def _readout_body(n_ref, p0_ref, p1_ref, Wa1_ref, Wa2_ref, ba_ref,
                  Wp1_ref, bp1_ref, Wp2_ref, bp2_ref, out_ref, acc_ref):
    i = pl.program_id(0)

    @pl.when(i == 0)
    def _():
        acc_ref[...] = jnp.zeros_like(acc_ref)

    agg = p0_ref[...] + p1_ref[...]
    hid = jnp.maximum(_dot(n_ref[...], Wa1_ref[...]) + _dot(agg, Wa2_ref[...])
                      + ba_ref[...], 0.0)
    acc_ref[0:1, :] += jnp.sum(hid, axis=0, keepdims=True)

    @pl.when(i == pl.num_programs(0) - 1)
    def _():
        g = acc_ref[0:1, :]
        z = jnp.maximum(_dot(g, Wp1_ref[...]) + bp1_ref[...], 0.0)
        out_ref[...] = _dot(z, Wp2_ref[...]) + bp2_ref[...]


def _full_spec(shape):
    return pl.BlockSpec(shape, lambda i: (0, 0))


def _prep_nodes(node, Wn, bn, Wh1):
    blk = 1000
    return pl.pallas_call(
        _prep_nodes_body,
        grid=(N // blk,),
        in_specs=[
            pl.BlockSpec((blk, H), lambda i: (i, 0)),
            _full_spec((H, H)), _full_spec((1, H)), _full_spec((H, H)),
        ],
        out_specs=[pl.BlockSpec((blk, H), lambda i: (i, 0))] * 2,
        out_shape=[jax.ShapeDtypeStruct((N, H), _f32)] * 2,
    )(node, Wn, bn, Wh1)


def _prep_edges(edge, We, be, Wh2, bh):
    blk = 2000
    nb = edge.shape[1]
    return pl.pallas_call(
        _prep_edges_body,
        grid=(E // blk,),
        in_specs=[
            pl.BlockSpec((blk, nb), lambda i: (i, 0)),
            _full_spec((nb, H)), _full_spec((1, H)),
            _full_spec((H, H)), _full_spec((1, H)),
        ],
        out_specs=pl.BlockSpec((blk, H), lambda i: (i, 0)),
        out_shape=jax.ShapeDtypeStruct((E, H), _f32),
    )(edge, We, be, Wh2, bh)


def _layer_tc(inputs, h, parts, Wmi, bmi):
    blk = 2000
    nblk = E // blk
    fblk = NP // nblk
    return pl.pallas_call(
        _layer_body,
        grid=(nblk,),
        in_specs=[
            pl.BlockSpec((blk, H), lambda i: (i, 0)),
            pl.BlockSpec((blk, H), lambda i: (i, 0)),
            pl.BlockSpec((fblk, H), lambda i: (i, 0)),
            pl.BlockSpec((fblk, H), lambda i: (i + NP // fblk, 0)),
            _full_spec((H, H)), _full_spec((1, H)),
        ],
        out_specs=[
            pl.BlockSpec((blk, H), lambda i: (i, 0)),
            pl.BlockSpec((fblk, H), lambda i: (i, 0)),
        ],
        out_shape=[
            jax.ShapeDtypeStruct((E, H), _f32),
            jax.ShapeDtypeStruct((NP, H), _f32),
        ],
    )(inputs, h, parts, parts, Wmi, bmi)


def _readout(n, parts, Wa1, Wa2, ba, Wp1, bp1, Wp2, bp2):
    blk = 80
    h2 = H // 2
    return pl.pallas_call(
        _readout_body,
        grid=(N // blk,),
        in_specs=[
            pl.BlockSpec((blk, H), lambda i: (i, 0)),
            pl.BlockSpec((blk, H), lambda i: (i, 0)),
            pl.BlockSpec((blk, H), lambda i: (i + NP // blk, 0)),
            _full_spec((H, H)), _full_spec((H, H)), _full_spec((1, H)),
            _full_spec((H, h2)), _full_spec((1, h2)),
            _full_spec((h2, 1)), _full_spec((1, 1)),
        ],
        out_specs=pl.BlockSpec((1, 1), lambda i: (0, 0)),
        out_shape=jax.ShapeDtypeStruct((1, 1), _f32),
        scratch_shapes=[pltpu.VMEM((8, H), _f32)],
    )(n, parts, parts, Wa1, Wa2, ba, Wp1, bp1, Wp2, bp2)


def kernel(node, edge, edge_index, Wn, bn, We, be, Wh, bh, Wa, ba,
           Wm, bm, Wp1, bp1, Wp2, bp2):
    src = edge_index[0].reshape(NCHUNK, CH)
    dst = edge_index[1].reshape(NCHUNK, CH)
    Wh1, Wh2 = Wh[:H], Wh[H:]
    Wa1, Wa2 = Wa[:H], Wa[H:]
    bn_, be_, bh_, ba_, bp1_, bp2_ = (
        x.reshape(1, -1) for x in (bn, be, bh, ba, bp1, bp2))
    zeros = jnp.zeros((NP, H), _f32)

    n, nWh = _prep_nodes(node, Wn, bn_, Wh1)
    eW = _prep_edges(edge, We, be_, Wh2, bh_)
    h, parts = _sc_pass(eW, src, dst, nWh, zeros)
    inputs = h
    for i in range(3):
        d, fullW = _layer_tc(inputs, h, parts, Wm[i], bm[i].reshape(1, -1))
        h, parts = _sc_pass(d, dst, dst, fullW, zeros)
    return _readout(n, parts, Wa1, Wa2, ba_, Wp1, bp1_, Wp2, bp2_)


# speedup vs baseline: 3.1565x; 1.0744x over previous
import functools

import jax
import jax.numpy as jnp
from jax import lax
from jax.experimental import pallas as pl
from jax.experimental.pallas import tpu as pltpu
from jax.experimental.pallas import tpu_sc as plsc

E = 320000
EH = E // 2
N = 10000
H = 128
CH = 64
NCHUNK = EH // CH
NC, NS = 2, 16
NW = NC * NS
CPT = 79
NP = 10240
ROWS_PER_TILE = NP // NS

_f32 = jnp.float32


@functools.partial(
    pl.kernel,
    mesh=plsc.VectorSubcoreMesh(core_axis_name="c", subcore_axis_name="s"),
    out_type=[
        jax.ShapeDtypeStruct((EH, H), _f32),
        jax.ShapeDtypeStruct((NC * NP, H), _f32),
    ],
    scratch_types=[
        pltpu.VMEM((CH, H), _f32), pltpu.VMEM((CH, H), _f32),
        pltpu.VMEM((CH, H), _f32), pltpu.VMEM((CH, H), _f32),
        pltpu.VMEM((CH,), jnp.int32), pltpu.VMEM((CH,), jnp.int32),
        pltpu.VMEM((CH,), jnp.int32), pltpu.VMEM((CH,), jnp.int32),
        pltpu.VMEM_SHARED((NP, H), _f32),
        pltpu.SemaphoreType.DMA, pltpu.SemaphoreType.DMA,
        pltpu.SemaphoreType.DMA, pltpu.SemaphoreType.DMA,
        pltpu.SemaphoreType.DMA, pltpu.SemaphoreType.DMA,
        pltpu.SemaphoreType.DMA, pltpu.SemaphoreType.DMA,
        pltpu.SemaphoreType.DMA, pltpu.SemaphoreType.DMA,
        pltpu.SemaphoreType.DMA, pltpu.SemaphoreType.DMA,
    ],
)
def _sc_pass(a_hbm, gidx_hbm, sidx_hbm, table_hbm, zeros_hbm,
             h_out, part_out,
             a0, a1, g0, g1, gi0, gi1, si0, si1, acc_sh,
             sa0, sa1, sg0, sg1, sgi0, sgi1, ssi0, ssi1,
             sh0, sh1, ss0, ss1):
    c = lax.axis_index("c")
    s = lax.axis_index("s")
    w = s * NC + c
    a_v, g_v = (a0, a1), (g0, g1)
    gi_v, si_v = (gi0, gi1), (si0, si1)
    sa, sg = (sa0, sa1), (sg0, sg1)
    sgi, ssi = (sgi0, sgi1), (ssi0, ssi1)
    sh, ss = (sh0, sh1), (ss0, ss1)

    def cid_of(j):
        return w + NW * j

    pltpu.sync_copy(zeros_hbm.at[pl.ds(s * ROWS_PER_TILE, ROWS_PER_TILE)],
                    acc_sh.at[pl.ds(s * ROWS_PER_TILE, ROWS_PER_TILE)])
    plsc.subcore_barrier()

    pltpu.async_copy(gidx_hbm.at[cid_of(0)], gi_v[0], sgi[0])

    @pl.when(cid_of(1) < NCHUNK)
    def _():
        pltpu.async_copy(gidx_hbm.at[cid_of(1)], gi_v[1], sgi[1])

    pltpu.async_copy(sidx_hbm.at[cid_of(0)], si_v[0], ssi[0])
    pltpu.make_async_copy(gidx_hbm.at[cid_of(0)], gi_v[0], sgi[0]).wait()
    pltpu.async_copy(a_hbm.at[pl.ds(cid_of(0) * CH, CH)], a_v[0], sa[0])
    pltpu.async_copy(table_hbm.at[gi_v[0]], g_v[0], sg[0])

    def pair_body(jj, carry):
        for b in range(2):
            j = 2 * jj + b
            o = 1 - b
            cid = cid_of(j)
            cprev = cid_of(j - 1)
            cnext = cid_of(j + 1)

            @pl.when(cid < NCHUNK)
            def _():
                pltpu.make_async_copy(
                    a_hbm.at[pl.ds(cid * CH, CH)], a_v[b], sa[b]).wait()
                pltpu.make_async_copy(
                    table_hbm.at[gi_v[b]], g_v[b], sg[b]).wait()

                def row_body(r, rc):
                    for q in range(H // 16):
                        sl = pl.ds(q * 16, 16)
                        a_v[b][r, sl] = jnp.maximum(
                            a_v[b][r, sl] + g_v[b][r, sl], 0.0)
                    return rc

                lax.fori_loop(0, CH, row_body, 0)
                pltpu.make_async_copy(
                    sidx_hbm.at[cid], si_v[b], ssi[b]).wait()
                pltpu.async_copy(a_v[b], h_out.at[pl.ds(cid * CH, CH)], sh[b])
                pltpu.async_copy(a_v[b], acc_sh.at[si_v[b]], ss[b], add=True)

            @pl.when(jnp.logical_and(j >= 1, cprev < NCHUNK))
            def _():
                pltpu.make_async_copy(
                    a_v[o], h_out.at[pl.ds(cprev * CH, CH)], sh[o]).wait()
                pltpu.make_async_copy(
                    a_v[o], acc_sh.at[si_v[o]], ss[o]).wait()

            @pl.when(cnext < NCHUNK)
            def _():
                pltpu.make_async_copy(
                    gidx_hbm.at[cnext], gi_v[o], sgi[o]).wait()
                pltpu.async_copy(
                    a_hbm.at[pl.ds(cnext * CH, CH)], a_v[o], sa[o])
                pltpu.async_copy(table_hbm.at[gi_v[o]], g_v[o], sg[o])
                pltpu.async_copy(sidx_hbm.at[cnext], si_v[o], ssi[o])

            @pl.when(cid_of(j + 2) < NCHUNK)
            def _():
                pltpu.async_copy(gidx_hbm.at[cid_of(j + 2)], gi_v[b], sgi[b])

        return carry

    lax.fori_loop(0, CPT // 2 + 1, pair_body, 0)
    plsc.subcore_barrier()
    pltpu.sync_copy(acc_sh.at[pl.ds(s * ROWS_PER_TILE, ROWS_PER_TILE)],
                    part_out.at[pl.ds(c * NP + s * ROWS_PER_TILE, ROWS_PER_TILE)])


def _dot(a, b):
    return jnp.dot(a, b, preferred_element_type=_f32)


def _leaky(x):
    return jnp.where(x > 0, x, 0.01 * x)


def _prep_nodes_body(node_ref, Wn_ref, bn_ref, Wh1_ref, n_ref, nWh_ref):
    n = _leaky(_dot(node_ref[...], Wn_ref[...]) + bn_ref[...])
    n_ref[...] = n
    nWh_ref[...] = _dot(n, Wh1_ref[...])


def _prep_edges_body(edge_ref, We_ref, be_ref, Wh2_ref, bh_ref, eW_ref):
    e = _leaky(_dot(edge_ref[...], We_ref[...]) + be_ref[...])
    eW_ref[...] = _dot(e, Wh2_ref[...]) + bh_ref[...]


def _layer_a_body(inp_ref, h_ref, pa0, pa1, pb0, pb1, Wm_ref, bm_ref,
                  d_ref, fw_ref):
    d_ref[...] = inp_ref[...] - _dot(h_ref[...], Wm_ref[...])
    full = pa0[...] + pa1[...] + pb0[...] + pb1[...]
    fw_ref[...] = _dot(full, Wm_ref[...]) + bm_ref[...]


def _layer_b_body(inp_ref, h_ref, Wm_ref, d_ref):
    d_ref[...] = inp_ref[...] - _dot(h_ref[...], Wm_ref[...])


def _readout_body(n_ref, pa0, pa1, pb0, pb1, Wa1_ref, Wa2_ref, ba_ref,
                  Wp1_ref, bp1_ref, Wp2_ref, bp2_ref, out_ref, acc_ref):
    i = pl.program_id(0)

    @pl.when(i == 0)
    def _():
        acc_ref[...] = jnp.zeros_like(acc_ref)

    agg = pa0[...] + pa1[...] + pb0[...] + pb1[...]
    hid = jnp.maximum(_dot(n_ref[...], Wa1_ref[...]) + _dot(agg, Wa2_ref[...])
                      + ba_ref[...], 0.0)
    acc_ref[0:1, :] += jnp.sum(hid, axis=0, keepdims=True)

    @pl.when(i == pl.num_programs(0) - 1)
    def _():
        g = acc_ref[0:1, :]
        z = jnp.maximum(_dot(g, Wp1_ref[...]) + bp1_ref[...], 0.0)
        out_ref[...] = _dot(z, Wp2_ref[...]) + bp2_ref[...]


def _full_spec(shape):
    return pl.BlockSpec(shape, lambda i: (0, 0))


def _prep_nodes(node, Wn, bn, Wh1):
    blk = 1000
    return pl.pallas_call(
        _prep_nodes_body,
        grid=(N // blk,),
        in_specs=[
            pl.BlockSpec((blk, H), lambda i: (i, 0)),
            _full_spec((H, H)), _full_spec((1, H)), _full_spec((H, H)),
        ],
        out_specs=[pl.BlockSpec((blk, H), lambda i: (i, 0))] * 2,
        out_shape=[jax.ShapeDtypeStruct((N, H), _f32)] * 2,
    )(node, Wn, bn, Wh1)


def _prep_edges(edge, We, be, Wh2, bh):
    blk = 2000
    nb = edge.shape[1]
    return pl.pallas_call(
        _prep_edges_body,
        grid=(EH // blk,),
        in_specs=[
            pl.BlockSpec((blk, nb), lambda i: (i, 0)),
            _full_spec((nb, H)), _full_spec((1, H)),
            _full_spec((H, H)), _full_spec((1, H)),
        ],
        out_specs=pl.BlockSpec((blk, H), lambda i: (i, 0)),
        out_shape=jax.ShapeDtypeStruct((EH, H), _f32),
    )(edge, We, be, Wh2, bh)


def _layer_tc_a(inputs, h, parts_a, parts_b, Wmi, bmi):
    blk = 2000
    nblk = EH // blk
    fblk = NP // nblk
    poff = NP // fblk
    return pl.pallas_call(
        _layer_a_body,
        grid=(nblk,),
        in_specs=[
            pl.BlockSpec((blk, H), lambda i: (i, 0)),
            pl.BlockSpec((blk, H), lambda i: (i, 0)),
            pl.BlockSpec((fblk, H), lambda i: (i, 0)),
            pl.BlockSpec((fblk, H), lambda i: (i + poff, 0)),
            pl.BlockSpec((fblk, H), lambda i: (i, 0)),
            pl.BlockSpec((fblk, H), lambda i: (i + poff, 0)),
            _full_spec((H, H)), _full_spec((1, H)),
        ],
        out_specs=[
            pl.BlockSpec((blk, H), lambda i: (i, 0)),
            pl.BlockSpec((fblk, H), lambda i: (i, 0)),
        ],
        out_shape=[
            jax.ShapeDtypeStruct((EH, H), _f32),
            jax.ShapeDtypeStruct((NP, H), _f32),
        ],
    )(inputs, h, parts_a, parts_a, parts_b, parts_b, Wmi, bmi)


def _layer_tc_b(inputs, h, Wmi):
    blk = 2000
    return pl.pallas_call(
        _layer_b_body,
        grid=(EH // blk,),
        in_specs=[
            pl.BlockSpec((blk, H), lambda i: (i, 0)),
            pl.BlockSpec((blk, H), lambda i: (i, 0)),
            _full_spec((H, H)),
        ],
        out_specs=pl.BlockSpec((blk, H), lambda i: (i, 0)),
        out_shape=jax.ShapeDtypeStruct((EH, H), _f32),
    )(inputs, h, Wmi)


def _readout(n, parts_a, parts_b, Wa1, Wa2, ba, Wp1, bp1, Wp2, bp2):
    blk = 80
    poff = NP // blk
    h2 = H // 2
    return pl.pallas_call(
        _readout_body,
        grid=(N // blk,),
        in_specs=[
            pl.BlockSpec((blk, H), lambda i: (i, 0)),
            pl.BlockSpec((blk, H), lambda i: (i, 0)),
            pl.BlockSpec((blk, H), lambda i: (i + poff, 0)),
            pl.BlockSpec((blk, H), lambda i: (i, 0)),
            pl.BlockSpec((blk, H), lambda i: (i + poff, 0)),
            _full_spec((H, H)), _full_spec((H, H)), _full_spec((1, H)),
            _full_spec((H, h2)), _full_spec((1, h2)),
            _full_spec((h2, 1)), _full_spec((1, 1)),
        ],
        out_specs=pl.BlockSpec((1, 1), lambda i: (0, 0)),
        out_shape=jax.ShapeDtypeStruct((1, 1), _f32),
        scratch_shapes=[pltpu.VMEM((8, H), _f32)],
    )(n, parts_a, parts_a, parts_b, parts_b, Wa1, Wa2, ba, Wp1, bp1, Wp2, bp2)


def kernel(node, edge, edge_index, Wn, bn, We, be, Wh, bh, Wa, ba,
           Wm, bm, Wp1, bp1, Wp2, bp2):
    src = edge_index[0].reshape(2, NCHUNK, CH)
    dst = edge_index[1].reshape(2, NCHUNK, CH)
    sA, sB = src[0], src[1]
    dA, dB = dst[0], dst[1]
    Wh1, Wh2 = Wh[:H], Wh[H:]
    Wa1, Wa2 = Wa[:H], Wa[H:]
    bn_, be_, bh_, ba_, bp1_, bp2_ = (
        x.reshape(1, -1) for x in (bn, be, bh, ba, bp1, bp2))
    zeros = jnp.zeros((NP, H), _f32)
    edge2 = edge.reshape(2, EH, -1)

    n, nWh = _prep_nodes(node, Wn, bn_, Wh1)
    eWA = _prep_edges(edge2[0], We, be_, Wh2, bh_)
    hA, pA = _sc_pass(eWA, sA, dA, nWh, zeros)
    eWB = _prep_edges(edge2[1], We, be_, Wh2, bh_)
    hB, pB = _sc_pass(eWB, sB, dB, nWh, zeros)
    inA, inB = hA, hB
    for i in range(3):
        Wmi, bmi = Wm[i], bm[i].reshape(1, -1)
        dA_, fullW = _layer_tc_a(inA, hA, pA, pB, Wmi, bmi)
        hA, pA = _sc_pass(dA_, dA, dA, fullW, zeros)
        dB_ = _layer_tc_b(inB, hB, Wmi)
        hB, pB = _sc_pass(dB_, dB, dB, fullW, zeros)
    return _readout(n, pA, pB, Wa1, Wa2, ba_, Wp1, bp1_, Wp2, bp2_)


# speedup vs baseline: 3.4327x; 1.0875x over previous
import functools

import jax
import jax.numpy as jnp
from jax import lax
from jax.experimental import pallas as pl
from jax.experimental.pallas import tpu as pltpu
from jax.experimental.pallas import tpu_sc as plsc

E = 320000
EH = E // 2
N = 10000
H = 128
CH = 64
NCHUNK = EH // CH
NC, NS = 2, 16
NW = NC * NS
CPT = 79
NP = 10240
ROWS_PER_TILE = NP // NS

_f32 = jnp.float32


@functools.partial(
    pl.kernel,
    mesh=plsc.VectorSubcoreMesh(core_axis_name="c", subcore_axis_name="s"),
    out_type=[
        jax.ShapeDtypeStruct((EH, H), _f32),
        jax.ShapeDtypeStruct((NC * NP, H), _f32),
    ],
    scratch_types=[
        pltpu.VMEM((CH, H), _f32), pltpu.VMEM((CH, H), _f32),
        pltpu.VMEM((CH, H), _f32), pltpu.VMEM((CH, H), _f32),
        pltpu.VMEM((CH,), jnp.int32), pltpu.VMEM((CH,), jnp.int32),
        pltpu.VMEM((CH,), jnp.int32), pltpu.VMEM((CH,), jnp.int32),
        pltpu.VMEM_SHARED((NP, H), _f32),
        pltpu.SemaphoreType.DMA, pltpu.SemaphoreType.DMA,
        pltpu.SemaphoreType.DMA, pltpu.SemaphoreType.DMA,
        pltpu.SemaphoreType.DMA, pltpu.SemaphoreType.DMA,
        pltpu.SemaphoreType.DMA, pltpu.SemaphoreType.DMA,
        pltpu.SemaphoreType.DMA, pltpu.SemaphoreType.DMA,
        pltpu.SemaphoreType.DMA, pltpu.SemaphoreType.DMA,
    ],
)
def _sc_pass(a_hbm, gidx_hbm, sidx_hbm, table_hbm, zeros_hbm,
             h_out, part_out,
             a0, a1, g0, g1, gi0, gi1, si0, si1, acc_sh,
             sa0, sa1, sg0, sg1, sgi0, sgi1, ssi0, ssi1,
             sh0, sh1, ss0, ss1):
    c = lax.axis_index("c")
    s = lax.axis_index("s")
    w = s * NC + c
    a_v, g_v = (a0, a1), (g0, g1)
    gi_v, si_v = (gi0, gi1), (si0, si1)
    sa, sg = (sa0, sa1), (sg0, sg1)
    sgi, ssi = (sgi0, sgi1), (ssi0, ssi1)
    sh, ss = (sh0, sh1), (ss0, ss1)

    def cid_of(j):
        return w + NW * j

    pltpu.sync_copy(zeros_hbm.at[pl.ds(s * ROWS_PER_TILE, ROWS_PER_TILE)],
                    acc_sh.at[pl.ds(s * ROWS_PER_TILE, ROWS_PER_TILE)])
    plsc.subcore_barrier()

    pltpu.async_copy(gidx_hbm.at[cid_of(0)], gi_v[0], sgi[0])

    @pl.when(cid_of(1) < NCHUNK)
    def _():
        pltpu.async_copy(gidx_hbm.at[cid_of(1)], gi_v[1], sgi[1])

    pltpu.async_copy(sidx_hbm.at[cid_of(0)], si_v[0], ssi[0])
    pltpu.make_async_copy(gidx_hbm.at[cid_of(0)], gi_v[0], sgi[0]).wait()
    pltpu.async_copy(a_hbm.at[pl.ds(cid_of(0) * CH, CH)], a_v[0], sa[0])
    pltpu.async_copy(table_hbm.at[gi_v[0]], g_v[0], sg[0])

    def pair_body(jj, carry):
        for b in range(2):
            j = 2 * jj + b
            o = 1 - b
            cid = cid_of(j)
            cprev = cid_of(j - 1)
            cnext = cid_of(j + 1)

            @pl.when(cid < NCHUNK)
            def _():
                pltpu.make_async_copy(
                    a_hbm.at[pl.ds(cid * CH, CH)], a_v[b], sa[b]).wait()
                pltpu.make_async_copy(
                    table_hbm.at[gi_v[b]], g_v[b], sg[b]).wait()

                def row_body(r, rc):
                    for q in range(H // 16):
                        sl = pl.ds(q * 16, 16)
                        a_v[b][r, sl] = jnp.maximum(
                            a_v[b][r, sl] + g_v[b][r, sl], 0.0)
                    return rc

                lax.fori_loop(0, CH, row_body, 0)
                pltpu.make_async_copy(
                    sidx_hbm.at[cid], si_v[b], ssi[b]).wait()
                pltpu.async_copy(a_v[b], h_out.at[pl.ds(cid * CH, CH)], sh[b])
                pltpu.async_copy(a_v[b], acc_sh.at[si_v[b]], ss[b], add=True)

            @pl.when(jnp.logical_and(j >= 1, cprev < NCHUNK))
            def _():
                pltpu.make_async_copy(
                    a_v[o], h_out.at[pl.ds(cprev * CH, CH)], sh[o]).wait()
                pltpu.make_async_copy(
                    a_v[o], acc_sh.at[si_v[o]], ss[o]).wait()

            @pl.when(cnext < NCHUNK)
            def _():
                pltpu.make_async_copy(
                    gidx_hbm.at[cnext], gi_v[o], sgi[o]).wait()
                pltpu.async_copy(
                    a_hbm.at[pl.ds(cnext * CH, CH)], a_v[o], sa[o])
                pltpu.async_copy(table_hbm.at[gi_v[o]], g_v[o], sg[o])
                pltpu.async_copy(sidx_hbm.at[cnext], si_v[o], ssi[o])

            @pl.when(cid_of(j + 2) < NCHUNK)
            def _():
                pltpu.async_copy(gidx_hbm.at[cid_of(j + 2)], gi_v[b], sgi[b])

        return carry

    lax.fori_loop(0, CPT // 2 + 1, pair_body, 0)
    plsc.subcore_barrier()
    pltpu.sync_copy(acc_sh.at[pl.ds(s * ROWS_PER_TILE, ROWS_PER_TILE)],
                    part_out.at[pl.ds(c * NP + s * ROWS_PER_TILE, ROWS_PER_TILE)])


def _dot(a, b):
    return jnp.dot(a, b, preferred_element_type=_f32)


def _leaky(x):
    return jnp.where(x > 0, x, 0.01 * x)


def _prep_nodes_body(node_ref, Wn_ref, bn_ref, Wh1_ref, n_ref, nWh_ref):
    n = _leaky(_dot(node_ref[...], Wn_ref[...]) + bn_ref[...])
    n_ref[...] = n
    nWh_ref[...] = _dot(n, Wh1_ref[...])


def _prep_edges_body(edge_ref, We_ref, be_ref, Wh2_ref, bh_ref, eW_ref):
    e = _leaky(_dot(edge_ref[...], We_ref[...]) + be_ref[...])
    eW_ref[...] = _dot(e, Wh2_ref[...]) + bh_ref[...]


def _fullw_body(pa0, pa1, pb0, pb1, Wm_ref, bm_ref, fw_ref):
    full = pa0[...] + pa1[...] + pb0[...] + pb1[...]
    fw_ref[...] = _dot(full, Wm_ref[...]) + bm_ref[...]


def _layer_b_body(inp_ref, h_ref, Wm_ref, d_ref):
    d_ref[...] = inp_ref[...] - _dot(h_ref[...], Wm_ref[...])


def _readout_body(n_ref, pa0, pa1, pb0, pb1, Wa1_ref, Wa2_ref, ba_ref,
                  Wp1_ref, bp1_ref, Wp2_ref, bp2_ref, out_ref, acc_ref):
    i = pl.program_id(0)

    @pl.when(i == 0)
    def _():
        acc_ref[...] = jnp.zeros_like(acc_ref)

    agg = pa0[...] + pa1[...] + pb0[...] + pb1[...]
    hid = jnp.maximum(_dot(n_ref[...], Wa1_ref[...]) + _dot(agg, Wa2_ref[...])
                      + ba_ref[...], 0.0)
    acc_ref[0:1, :] += jnp.sum(hid, axis=0, keepdims=True)

    @pl.when(i == pl.num_programs(0) - 1)
    def _():
        g = acc_ref[0:1, :]
        z = jnp.maximum(_dot(g, Wp1_ref[...]) + bp1_ref[...], 0.0)
        out_ref[...] = _dot(z, Wp2_ref[...]) + bp2_ref[...]


def _full_spec(shape):
    return pl.BlockSpec(shape, lambda i: (0, 0))


def _prep_nodes(node, Wn, bn, Wh1):
    blk = 1000
    return pl.pallas_call(
        _prep_nodes_body,
        grid=(N // blk,),
        in_specs=[
            pl.BlockSpec((blk, H), lambda i: (i, 0)),
            _full_spec((H, H)), _full_spec((1, H)), _full_spec((H, H)),
        ],
        out_specs=[pl.BlockSpec((blk, H), lambda i: (i, 0))] * 2,
        out_shape=[jax.ShapeDtypeStruct((N, H), _f32)] * 2,
    )(node, Wn, bn, Wh1)


def _prep_edges(edge, We, be, Wh2, bh):
    blk = 2000
    nb = edge.shape[1]
    return pl.pallas_call(
        _prep_edges_body,
        grid=(EH // blk,),
        in_specs=[
            pl.BlockSpec((blk, nb), lambda i: (i, 0)),
            _full_spec((nb, H)), _full_spec((1, H)),
            _full_spec((H, H)), _full_spec((1, H)),
        ],
        out_specs=pl.BlockSpec((blk, H), lambda i: (i, 0)),
        out_shape=jax.ShapeDtypeStruct((EH, H), _f32),
    )(edge, We, be, Wh2, bh)


def _fullw_tc(parts_a, parts_b, Wmi, bmi):
    blk = 1024
    poff = NP // blk
    return pl.pallas_call(
        _fullw_body,
        grid=(NP // blk,),
        in_specs=[
            pl.BlockSpec((blk, H), lambda i: (i, 0)),
            pl.BlockSpec((blk, H), lambda i: (i + poff, 0)),
            pl.BlockSpec((blk, H), lambda i: (i, 0)),
            pl.BlockSpec((blk, H), lambda i: (i + poff, 0)),
            _full_spec((H, H)), _full_spec((1, H)),
        ],
        out_specs=pl.BlockSpec((blk, H), lambda i: (i, 0)),
        out_shape=jax.ShapeDtypeStruct((NP, H), _f32),
    )(parts_a, parts_a, parts_b, parts_b, Wmi, bmi)


def _layer_tc_b(inputs, h, Wmi):
    blk = 2000
    return pl.pallas_call(
        _layer_b_body,
        grid=(EH // blk,),
        in_specs=[
            pl.BlockSpec((blk, H), lambda i: (i, 0)),
            pl.BlockSpec((blk, H), lambda i: (i, 0)),
            _full_spec((H, H)),
        ],
        out_specs=pl.BlockSpec((blk, H), lambda i: (i, 0)),
        out_shape=jax.ShapeDtypeStruct((EH, H), _f32),
    )(inputs, h, Wmi)


def _readout(n, parts_a, parts_b, Wa1, Wa2, ba, Wp1, bp1, Wp2, bp2):
    blk = 80
    poff = NP // blk
    h2 = H // 2
    return pl.pallas_call(
        _readout_body,
        grid=(N // blk,),
        in_specs=[
            pl.BlockSpec((blk, H), lambda i: (i, 0)),
            pl.BlockSpec((blk, H), lambda i: (i, 0)),
            pl.BlockSpec((blk, H), lambda i: (i + poff, 0)),
            pl.BlockSpec((blk, H), lambda i: (i, 0)),
            pl.BlockSpec((blk, H), lambda i: (i + poff, 0)),
            _full_spec((H, H)), _full_spec((H, H)), _full_spec((1, H)),
            _full_spec((H, h2)), _full_spec((1, h2)),
            _full_spec((h2, 1)), _full_spec((1, 1)),
        ],
        out_specs=pl.BlockSpec((1, 1), lambda i: (0, 0)),
        out_shape=jax.ShapeDtypeStruct((1, 1), _f32),
        scratch_shapes=[pltpu.VMEM((8, H), _f32)],
    )(n, parts_a, parts_a, parts_b, parts_b, Wa1, Wa2, ba, Wp1, bp1, Wp2, bp2)


def kernel(node, edge, edge_index, Wn, bn, We, be, Wh, bh, Wa, ba,
           Wm, bm, Wp1, bp1, Wp2, bp2):
    src = edge_index[0].reshape(2, NCHUNK, CH)
    dst = edge_index[1].reshape(2, NCHUNK, CH)
    sA, sB = src[0], src[1]
    dA, dB = dst[0], dst[1]
    Wh1, Wh2 = Wh[:H], Wh[H:]
    Wa1, Wa2 = Wa[:H], Wa[H:]
    bn_, be_, bh_, ba_, bp1_, bp2_ = (
        x.reshape(1, -1) for x in (bn, be, bh, ba, bp1, bp2))
    zeros = jnp.zeros((NP, H), _f32)
    edge2 = edge.reshape(2, EH, -1)

    n, nWh = _prep_nodes(node, Wn, bn_, Wh1)
    eWA = _prep_edges(edge2[0], We, be_, Wh2, bh_)
    hA, pA = _sc_pass(eWA, sA, dA, nWh, zeros)
    eWB = _prep_edges(edge2[1], We, be_, Wh2, bh_)
    hB, pB = _sc_pass(eWB, sB, dB, nWh, zeros)
    inA, inB = hA, hB
    dA_ = _layer_tc_b(inA, hA, Wm[0])
    for i in range(3):
        Wmi, bmi = Wm[i], bm[i].reshape(1, -1)
        fullW = _fullw_tc(pA, pB, Wmi, bmi)
        hA, pA = _sc_pass(dA_, dA, dA, fullW, zeros)
        dB_ = _layer_tc_b(inB, hB, Wmi)
        hB, pB = _sc_pass(dB_, dB, dB, fullW, zeros)
        if i < 2:
            dA_ = _layer_tc_b(inA, hA, Wm[i + 1])
    return _readout(n, pA, pB, Wa1, Wa2, ba_, Wp1, bp1_, Wp2, bp2_)


# speedup vs baseline: 3.5733x; 1.0409x over previous
import functools

import jax
import jax.numpy as jnp
from jax import lax
from jax.experimental import pallas as pl
from jax.experimental.pallas import tpu as pltpu
from jax.experimental.pallas import tpu_sc as plsc

E = 320000
EH = E // 2
N = 10000
H = 128
CH = 80
NCHUNK = EH // CH
NC, NS = 2, 16
NW = NC * NS
CPT = 63
NP = 10240
ROWS_PER_TILE = NP // NS

_f32 = jnp.float32


@functools.partial(
    pl.kernel,
    mesh=plsc.VectorSubcoreMesh(core_axis_name="c", subcore_axis_name="s"),
    out_type=[
        jax.ShapeDtypeStruct((EH, H), _f32),
        jax.ShapeDtypeStruct((NC * NP, H), _f32),
    ],
    scratch_types=[
        pltpu.VMEM((CH, H), _f32), pltpu.VMEM((CH, H), _f32),
        pltpu.VMEM((CH, H), _f32), pltpu.VMEM((CH, H), _f32),
        pltpu.VMEM((CH,), jnp.int32), pltpu.VMEM((CH,), jnp.int32),
        pltpu.VMEM((CH,), jnp.int32), pltpu.VMEM((CH,), jnp.int32),
        pltpu.VMEM_SHARED((NP, H), _f32),
        pltpu.SemaphoreType.DMA, pltpu.SemaphoreType.DMA,
        pltpu.SemaphoreType.DMA, pltpu.SemaphoreType.DMA,
        pltpu.SemaphoreType.DMA, pltpu.SemaphoreType.DMA,
        pltpu.SemaphoreType.DMA, pltpu.SemaphoreType.DMA,
        pltpu.SemaphoreType.DMA, pltpu.SemaphoreType.DMA,
        pltpu.SemaphoreType.DMA, pltpu.SemaphoreType.DMA,
    ],
)
def _sc_pass(a_hbm, gidx_hbm, sidx_hbm, table_hbm, zeros_hbm,
             h_out, part_out,
             a0, a1, g0, g1, gi0, gi1, si0, si1, acc_sh,
             sa0, sa1, sg0, sg1, sgi0, sgi1, ssi0, ssi1,
             sh0, sh1, ss0, ss1):
    c = lax.axis_index("c")
    s = lax.axis_index("s")
    w = s * NC + c
    a_v, g_v = (a0, a1), (g0, g1)
    gi_v, si_v = (gi0, gi1), (si0, si1)
    sa, sg = (sa0, sa1), (sg0, sg1)
    sgi, ssi = (sgi0, sgi1), (ssi0, ssi1)
    sh, ss = (sh0, sh1), (ss0, ss1)

    def cid_of(j):
        return w + NW * j

    pltpu.sync_copy(zeros_hbm.at[pl.ds(s * ROWS_PER_TILE, ROWS_PER_TILE)],
                    acc_sh.at[pl.ds(s * ROWS_PER_TILE, ROWS_PER_TILE)])
    plsc.subcore_barrier()

    pltpu.async_copy(gidx_hbm.at[cid_of(0)], gi_v[0], sgi[0])

    @pl.when(cid_of(1) < NCHUNK)
    def _():
        pltpu.async_copy(gidx_hbm.at[cid_of(1)], gi_v[1], sgi[1])

    pltpu.async_copy(sidx_hbm.at[cid_of(0)], si_v[0], ssi[0])
    pltpu.make_async_copy(gidx_hbm.at[cid_of(0)], gi_v[0], sgi[0]).wait()
    pltpu.async_copy(a_hbm.at[pl.ds(cid_of(0) * CH, CH)], a_v[0], sa[0])
    pltpu.async_copy(table_hbm.at[gi_v[0]], g_v[0], sg[0])

    def pair_body(jj, carry):
        for b in range(2):
            j = 2 * jj + b
            o = 1 - b
            cid = cid_of(j)
            cprev = cid_of(j - 1)
            cnext = cid_of(j + 1)

            @pl.when(cid < NCHUNK)
            def _():
                pltpu.make_async_copy(
                    a_hbm.at[pl.ds(cid * CH, CH)], a_v[b], sa[b]).wait()
                pltpu.make_async_copy(
                    table_hbm.at[gi_v[b]], g_v[b], sg[b]).wait()

                def row_body(r, rc):
                    for q in range(H // 16):
                        sl = pl.ds(q * 16, 16)
                        a_v[b][r, sl] = jnp.maximum(
                            a_v[b][r, sl] + g_v[b][r, sl], 0.0)
                    return rc

                lax.fori_loop(0, CH, row_body, 0)
                pltpu.make_async_copy(
                    sidx_hbm.at[cid], si_v[b], ssi[b]).wait()
                pltpu.async_copy(a_v[b], h_out.at[pl.ds(cid * CH, CH)], sh[b])
                pltpu.async_copy(a_v[b], acc_sh.at[si_v[b]], ss[b], add=True)

            @pl.when(jnp.logical_and(j >= 1, cprev < NCHUNK))
            def _():
                pltpu.make_async_copy(
                    a_v[o], h_out.at[pl.ds(cprev * CH, CH)], sh[o]).wait()
                pltpu.make_async_copy(
                    a_v[o], acc_sh.at[si_v[o]], ss[o]).wait()

            @pl.when(cnext < NCHUNK)
            def _():
                pltpu.make_async_copy(
                    gidx_hbm.at[cnext], gi_v[o], sgi[o]).wait()
                pltpu.async_copy(
                    a_hbm.at[pl.ds(cnext * CH, CH)], a_v[o], sa[o])
                pltpu.async_copy(table_hbm.at[gi_v[o]], g_v[o], sg[o])
                pltpu.async_copy(sidx_hbm.at[cnext], si_v[o], ssi[o])

            @pl.when(cid_of(j + 2) < NCHUNK)
            def _():
                pltpu.async_copy(gidx_hbm.at[cid_of(j + 2)], gi_v[b], sgi[b])

        return carry

    lax.fori_loop(0, CPT // 2 + 1, pair_body, 0)
    plsc.subcore_barrier()
    pltpu.sync_copy(acc_sh.at[pl.ds(s * ROWS_PER_TILE, ROWS_PER_TILE)],
                    part_out.at[pl.ds(c * NP + s * ROWS_PER_TILE, ROWS_PER_TILE)])


def _dot(a, b):
    return jnp.dot(a, b, preferred_element_type=_f32)


def _leaky(x):
    return jnp.where(x > 0, x, 0.01 * x)


def _prep_nodes_body(node_ref, Wn_ref, bn_ref, Wh1_ref, n_ref, nWh_ref):
    n = _leaky(_dot(node_ref[...], Wn_ref[...]) + bn_ref[...])
    n_ref[...] = n
    nWh_ref[...] = _dot(n, Wh1_ref[...])


def _prep_edges_body(edge_ref, We_ref, be_ref, Wh2_ref, bh_ref, eW_ref):
    e = _leaky(_dot(edge_ref[...], We_ref[...]) + be_ref[...])
    eW_ref[...] = _dot(e, Wh2_ref[...]) + bh_ref[...]


def _fullw_body(pa0, pa1, pb0, pb1, Wm_ref, bm_ref, fw_ref):
    full = pa0[...] + pa1[...] + pb0[...] + pb1[...]
    fw_ref[...] = _dot(full, Wm_ref[...]) + bm_ref[...]


def _layer_b_body(inp_ref, h_ref, Wm_ref, d_ref):
    d_ref[...] = inp_ref[...] - _dot(h_ref[...], Wm_ref[...])


def _readout_body(n_ref, pa0, pa1, pb0, pb1, Wa1_ref, Wa2_ref, ba_ref,
                  Wp1_ref, bp1_ref, Wp2_ref, bp2_ref, out_ref, acc_ref):
    i = pl.program_id(0)

    @pl.when(i == 0)
    def _():
        acc_ref[...] = jnp.zeros_like(acc_ref)

    agg = pa0[...] + pa1[...] + pb0[...] + pb1[...]
    hid = jnp.maximum(_dot(n_ref[...], Wa1_ref[...]) + _dot(agg, Wa2_ref[...])
                      + ba_ref[...], 0.0)
    acc_ref[0:1, :] += jnp.sum(hid, axis=0, keepdims=True)

    @pl.when(i == pl.num_programs(0) - 1)
    def _():
        g = acc_ref[0:1, :]
        z = jnp.maximum(_dot(g, Wp1_ref[...]) + bp1_ref[...], 0.0)
        out_ref[...] = _dot(z, Wp2_ref[...]) + bp2_ref[...]


def _full_spec(shape):
    return pl.BlockSpec(shape, lambda i: (0, 0))


def _prep_nodes(node, Wn, bn, Wh1):
    blk = 1000
    return pl.pallas_call(
        _prep_nodes_body,
        grid=(N // blk,),
        in_specs=[
            pl.BlockSpec((blk, H), lambda i: (i, 0)),
            _full_spec((H, H)), _full_spec((1, H)), _full_spec((H, H)),
        ],
        out_specs=[pl.BlockSpec((blk, H), lambda i: (i, 0))] * 2,
        out_shape=[jax.ShapeDtypeStruct((N, H), _f32)] * 2,
    )(node, Wn, bn, Wh1)


def _prep_edges(edge, We, be, Wh2, bh):
    blk = 2000
    nb = edge.shape[1]
    return pl.pallas_call(
        _prep_edges_body,
        grid=(EH // blk,),
        in_specs=[
            pl.BlockSpec((blk, nb), lambda i: (i, 0)),
            _full_spec((nb, H)), _full_spec((1, H)),
            _full_spec((H, H)), _full_spec((1, H)),
        ],
        out_specs=pl.BlockSpec((blk, H), lambda i: (i, 0)),
        out_shape=jax.ShapeDtypeStruct((EH, H), _f32),
    )(edge, We, be, Wh2, bh)


def _fullw_tc(parts_a, parts_b, Wmi, bmi):
    blk = 1024
    poff = NP // blk
    return pl.pallas_call(
        _fullw_body,
        grid=(NP // blk,),
        in_specs=[
            pl.BlockSpec((blk, H), lambda i: (i, 0)),
            pl.BlockSpec((blk, H), lambda i: (i + poff, 0)),
            pl.BlockSpec((blk, H), lambda i: (i, 0)),
            pl.BlockSpec((blk, H), lambda i: (i + poff, 0)),
            _full_spec((H, H)), _full_spec((1, H)),
        ],
        out_specs=pl.BlockSpec((blk, H), lambda i: (i, 0)),
        out_shape=jax.ShapeDtypeStruct((NP, H), _f32),
    )(parts_a, parts_a, parts_b, parts_b, Wmi, bmi)


def _layer_tc_b(inputs, h, Wmi):
    blk = 2000
    return pl.pallas_call(
        _layer_b_body,
        grid=(EH // blk,),
        in_specs=[
            pl.BlockSpec((blk, H), lambda i: (i, 0)),
            pl.BlockSpec((blk, H), lambda i: (i, 0)),
            _full_spec((H, H)),
        ],
        out_specs=pl.BlockSpec((blk, H), lambda i: (i, 0)),
        out_shape=jax.ShapeDtypeStruct((EH, H), _f32),
    )(inputs, h, Wmi)


def _readout(n, parts_a, parts_b, Wa1, Wa2, ba, Wp1, bp1, Wp2, bp2):
    blk = 80
    poff = NP // blk
    h2 = H // 2
    return pl.pallas_call(
        _readout_body,
        grid=(N // blk,),
        in_specs=[
            pl.BlockSpec((blk, H), lambda i: (i, 0)),
            pl.BlockSpec((blk, H), lambda i: (i, 0)),
            pl.BlockSpec((blk, H), lambda i: (i + poff, 0)),
            pl.BlockSpec((blk, H), lambda i: (i, 0)),
            pl.BlockSpec((blk, H), lambda i: (i + poff, 0)),
            _full_spec((H, H)), _full_spec((H, H)), _full_spec((1, H)),
            _full_spec((H, h2)), _full_spec((1, h2)),
            _full_spec((h2, 1)), _full_spec((1, 1)),
        ],
        out_specs=pl.BlockSpec((1, 1), lambda i: (0, 0)),
        out_shape=jax.ShapeDtypeStruct((1, 1), _f32),
        scratch_shapes=[pltpu.VMEM((8, H), _f32)],
    )(n, parts_a, parts_a, parts_b, parts_b, Wa1, Wa2, ba, Wp1, bp1, Wp2, bp2)


def kernel(node, edge, edge_index, Wn, bn, We, be, Wh, bh, Wa, ba,
           Wm, bm, Wp1, bp1, Wp2, bp2):
    src = edge_index[0].reshape(2, NCHUNK, CH)
    dst = edge_index[1].reshape(2, NCHUNK, CH)
    sA, sB = src[0], src[1]
    dA, dB = dst[0], dst[1]
    Wh1, Wh2 = Wh[:H], Wh[H:]
    Wa1, Wa2 = Wa[:H], Wa[H:]
    bn_, be_, bh_, ba_, bp1_, bp2_ = (
        x.reshape(1, -1) for x in (bn, be, bh, ba, bp1, bp2))
    zeros = jnp.zeros((NP, H), _f32)
    edge2 = edge.reshape(2, EH, -1)

    n, nWh = _prep_nodes(node, Wn, bn_, Wh1)
    eWA = _prep_edges(edge2[0], We, be_, Wh2, bh_)
    hA, pA = _sc_pass(eWA, sA, dA, nWh, zeros)
    eWB = _prep_edges(edge2[1], We, be_, Wh2, bh_)
    hB, pB = _sc_pass(eWB, sB, dB, nWh, zeros)
    inA, inB = hA, hB
    dA_ = _layer_tc_b(inA, hA, Wm[0])
    for i in range(3):
        Wmi, bmi = Wm[i], bm[i].reshape(1, -1)
        fullW = _fullw_tc(pA, pB, Wmi, bmi)
        hA, pA = _sc_pass(dA_, dA, dA, fullW, zeros)
        dB_ = _layer_tc_b(inB, hB, Wmi)
        hB, pB = _sc_pass(dB_, dB, dB, fullW, zeros)
        if i < 2:
            dA_ = _layer_tc_b(inA, hA, Wm[i + 1])
    return _readout(n, pA, pB, Wa1, Wa2, ba_, Wp1, bp1_, Wp2, bp2_)
